# baseline (speedup 1.0000x reference)
import functools

import jax
import jax.numpy as jnp
from jax import lax
from jax.experimental import pallas as pl
from jax.experimental.pallas import tpu as pltpu
from jax.experimental.pallas import tpu_sc as plsc

NF = 64
NB = 100
NBP = 112
BATCH = 2048
LANE = 256
NC, NS, L = 2, 16, 16
NW = NC * NS
BPW = BATCH // NW
CH = 2
NCHUNK = BPW // CH


_TC_CHUNK = 256


def _minmax_body(x_ref, mn2_ref, inv_ref, mn_ref, mx_ref):
    i = pl.program_id(0)
    blk = x_ref[...]
    bmn = jnp.min(blk, axis=(0, 2)).reshape(NF, 1)
    bmx = jnp.max(blk, axis=(0, 2)).reshape(NF, 1)

    @pl.when(i == 0)
    def _():
        mn_ref[...] = bmn
        mx_ref[...] = bmx

    @pl.when(i > 0)
    def _():
        mn_ref[...] = jnp.minimum(mn_ref[...], bmn)
        mx_ref[...] = jnp.maximum(mx_ref[...], bmx)

    @pl.when(i == pl.num_programs(0) - 1)
    def _():
        mn = mn_ref[...]
        w = (mx_ref[...] - mn) / NB
        frow = lax.broadcasted_iota(
            jnp.int32, (NF, 1), 0).astype(jnp.float32) * float(NBP)
        mn2 = mn - frow * w
        inv_ref[...] = jnp.broadcast_to(1.0 / w, (NF, L))
        mn2_ref[...] = jnp.broadcast_to(mn2, (NF, L))


def _minmax(x):
    return pl.pallas_call(
        _minmax_body,
        grid=(BATCH // _TC_CHUNK,),
        in_specs=[pl.BlockSpec((_TC_CHUNK, NF, LANE), lambda i: (i, 0, 0))],
        out_specs=[
            pl.BlockSpec((NF, L), lambda i: (0, 0)),
            pl.BlockSpec((NF, L), lambda i: (0, 0)),
        ],
        out_shape=[
            jax.ShapeDtypeStruct((NF, L), jnp.float32),
            jax.ShapeDtypeStruct((NF, L), jnp.float32),
        ],
        scratch_shapes=[
            pltpu.VMEM((NF, 1), jnp.float32),
            pltpu.VMEM((NF, 1), jnp.float32),
        ],
    )(x)


_sc_mesh = plsc.VectorSubcoreMesh(
    core_axis_name="c", subcore_axis_name="s", num_cores=NC, num_subcores=NS)


@functools.partial(
    pl.kernel,
    out_type=jax.ShapeDtypeStruct((NW, NF * NBP), jnp.float32),
    mesh=_sc_mesh,
    compiler_params=pltpu.CompilerParams(needs_layout_passes=False),
    scratch_types=[
        pltpu.VMEM((CH, NF, LANE), jnp.float32),
        pltpu.VMEM((CH, NF, LANE), jnp.float32),
        pltpu.VMEM((NF * NBP,), jnp.float32),
        pltpu.VMEM((NF, L), jnp.float32),
        pltpu.VMEM((NF, L), jnp.float32),
        pltpu.SemaphoreType.DMA,
        pltpu.SemaphoreType.DMA,
    ],
)
def _sc_hist(x_hbm, mn_hbm, inv_hbm, out_hbm,
             buf_a, buf_b, hist, mn_v, inv_v, sem_a, sem_b):
    wid = lax.axis_index("s") * NC + lax.axis_index("c")
    base = wid * BPW

    pltpu.sync_copy(mn_hbm, mn_v)
    pltpu.sync_copy(inv_hbm, inv_v)

    zeros16 = jnp.zeros((L,), jnp.float32)
    ones16 = jnp.ones((L,), jnp.float32)

    def _zero_row(r, _):
        hist[pl.ds(r * L, L)] = zeros16
        return 0

    lax.fori_loop(0, NF * NBP // L, _zero_row, 0)

    def _chunk_copy(ci, buf, sem):
        return pltpu.make_async_copy(
            x_hbm.at[pl.ds(base + ci * CH, CH)], buf, sem)

    def _process(buf):
        def _row(r, _):
            bb = r // (NF // 2)
            f0 = r - bb * (NF // 2)
            f1 = f0 + NF // 2
            mnv0 = mn_v[f0, :]
            invv0 = inv_v[f0, :]
            mnv1 = mn_v[f1, :]
            invv1 = inv_v[f1, :]

            @plsc.parallel_loop(0, LANE, step=L, unroll=LANE // L)
            def _k(off):
                v0 = buf[bb, f0, pl.ds(off, L)]
                b0 = ((v0 - mnv0) * invv0).astype(jnp.int32)
                plsc.addupdate_scatter(hist, [b0], ones16)
                v1 = buf[bb, f1, pl.ds(off, L)]
                b1 = ((v1 - mnv1) * invv1).astype(jnp.int32)
                plsc.addupdate_scatter(hist, [b1], ones16)

            return 0

        lax.fori_loop(0, CH * NF // 2, _row, 0)

    _chunk_copy(0, buf_a, sem_a).start()

    def _outer(g, _):
        c0 = 2 * g
        _chunk_copy(c0, buf_a, sem_a).wait()
        _chunk_copy(c0 + 1, buf_b, sem_b).start()
        _process(buf_a)
        _chunk_copy(c0 + 1, buf_b, sem_b).wait()

        @pl.when(c0 + 2 < NCHUNK)
        def _():
            _chunk_copy(c0 + 2, buf_a, sem_a).start()

        _process(buf_b)
        return 0

    lax.fori_loop(0, NCHUNK // 2, _outer, 0)

    pltpu.sync_copy(hist, out_hbm.at[wid])


def _fold_body(p_ref, o_ref):
    h = jnp.sum(p_ref[...], axis=0)
    kcol = lax.broadcasted_iota(jnp.int32, (NF, NB), 1)
    o_ref[...] = h[:, :NB] + jnp.where(kcol == NB - 1, h[:, NB:NB + 1], 0.0)


def _fold(parts):
    return pl.pallas_call(
        _fold_body,
        out_shape=jax.ShapeDtypeStruct((NF, NB), jnp.float32),
    )(parts)


def kernel(x):
    mn2_sp, inv_sp = _minmax(x)
    parts = _sc_hist(x, mn2_sp, inv_sp)
    return _fold(parts.reshape(NW, NF, NBP))


# speedup vs baseline: 1.0046x; 1.0046x over previous
import functools

import jax
import jax.numpy as jnp
from jax import lax
from jax.experimental import pallas as pl
from jax.experimental.pallas import tpu as pltpu
from jax.experimental.pallas import tpu_sc as plsc

NF = 64
NB = 100
NBP = 112
BATCH = 2048
LANE = 256
NC, NS, L = 2, 16, 16
NW = NC * NS
BPW = BATCH // NW
CH = 2
NCHUNK = BPW // CH


_TC_CHUNK = 128


def _minmax_body(x_ref, mn2_ref, inv_ref, mn_ref, mx_ref):
    i = pl.program_id(0)
    blk = x_ref[...]
    bmn = jnp.min(blk, axis=(0, 2)).reshape(NF, 1)
    bmx = jnp.max(blk, axis=(0, 2)).reshape(NF, 1)

    @pl.when(i == 0)
    def _():
        mn_ref[...] = bmn
        mx_ref[...] = bmx

    @pl.when(i > 0)
    def _():
        mn_ref[...] = jnp.minimum(mn_ref[...], bmn)
        mx_ref[...] = jnp.maximum(mx_ref[...], bmx)

    @pl.when(i == pl.num_programs(0) - 1)
    def _():
        mn = mn_ref[...]
        w = (mx_ref[...] - mn) / NB
        frow = lax.broadcasted_iota(
            jnp.int32, (NF, 1), 0).astype(jnp.float32) * float(NBP)
        mn2 = mn - frow * w
        inv_ref[...] = jnp.broadcast_to(1.0 / w, (NF, L))
        mn2_ref[...] = jnp.broadcast_to(mn2, (NF, L))


def _minmax(x):
    return pl.pallas_call(
        _minmax_body,
        grid=(BATCH // _TC_CHUNK,),
        in_specs=[pl.BlockSpec((_TC_CHUNK, NF, LANE), lambda i: (i, 0, 0))],
        out_specs=[
            pl.BlockSpec((NF, L), lambda i: (0, 0)),
            pl.BlockSpec((NF, L), lambda i: (0, 0)),
        ],
        out_shape=[
            jax.ShapeDtypeStruct((NF, L), jnp.float32),
            jax.ShapeDtypeStruct((NF, L), jnp.float32),
        ],
        scratch_shapes=[
            pltpu.VMEM((NF, 1), jnp.float32),
            pltpu.VMEM((NF, 1), jnp.float32),
        ],
    )(x)


_sc_mesh = plsc.VectorSubcoreMesh(
    core_axis_name="c", subcore_axis_name="s", num_cores=NC, num_subcores=NS)


@functools.partial(
    pl.kernel,
    out_type=jax.ShapeDtypeStruct((NW, NF * NBP), jnp.float32),
    mesh=_sc_mesh,
    compiler_params=pltpu.CompilerParams(needs_layout_passes=False),
    scratch_types=[
        pltpu.VMEM((CH, NF, LANE), jnp.float32),
        pltpu.VMEM((CH, NF, LANE), jnp.float32),
        pltpu.VMEM((NF * NBP,), jnp.float32),
        pltpu.VMEM((NF, L), jnp.float32),
        pltpu.VMEM((NF, L), jnp.float32),
        pltpu.SemaphoreType.DMA,
        pltpu.SemaphoreType.DMA,
    ],
)
def _sc_hist(x_hbm, mn_hbm, inv_hbm, out_hbm,
             buf_a, buf_b, hist, mn_v, inv_v, sem_a, sem_b):
    wid = lax.axis_index("s") * NC + lax.axis_index("c")
    base = wid * BPW

    pltpu.sync_copy(mn_hbm, mn_v)
    pltpu.sync_copy(inv_hbm, inv_v)

    zeros16 = jnp.zeros((L,), jnp.float32)
    ones16 = jnp.ones((L,), jnp.float32)

    def _zero_row(r, _):
        hist[pl.ds(r * L, L)] = zeros16
        return 0

    lax.fori_loop(0, NF * NBP // L, _zero_row, 0)

    def _chunk_copy(ci, buf, sem):
        return pltpu.make_async_copy(
            x_hbm.at[pl.ds(base + ci * CH, CH)], buf, sem)

    def _process(buf):
        def _row(r, _):
            bb = r // (NF // 2)
            f0 = r - bb * (NF // 2)
            f1 = f0 + NF // 2
            mnv0 = mn_v[f0, :]
            invv0 = inv_v[f0, :]
            mnv1 = mn_v[f1, :]
            invv1 = inv_v[f1, :]

            @plsc.parallel_loop(0, LANE, step=L, unroll=LANE // L)
            def _k(off):
                v0 = buf[bb, f0, pl.ds(off, L)]
                b0 = ((v0 - mnv0) * invv0).astype(jnp.int32)
                plsc.addupdate_scatter(hist, [b0], ones16)
                v1 = buf[bb, f1, pl.ds(off, L)]
                b1 = ((v1 - mnv1) * invv1).astype(jnp.int32)
                plsc.addupdate_scatter(hist, [b1], ones16)

            return 0

        lax.fori_loop(0, CH * NF // 2, _row, 0)

    _chunk_copy(0, buf_a, sem_a).start()

    def _outer(g, _):
        c0 = 2 * g
        _chunk_copy(c0, buf_a, sem_a).wait()
        _chunk_copy(c0 + 1, buf_b, sem_b).start()
        _process(buf_a)
        _chunk_copy(c0 + 1, buf_b, sem_b).wait()

        @pl.when(c0 + 2 < NCHUNK)
        def _():
            _chunk_copy(c0 + 2, buf_a, sem_a).start()

        _process(buf_b)
        return 0

    lax.fori_loop(0, NCHUNK // 2, _outer, 0)

    pltpu.sync_copy(hist, out_hbm.at[wid])


def _fold_body(p_ref, o_ref):
    h = jnp.sum(p_ref[...], axis=0)
    kcol = lax.broadcasted_iota(jnp.int32, (NF, NB), 1)
    o_ref[...] = h[:, :NB] + jnp.where(kcol == NB - 1, h[:, NB:NB + 1], 0.0)


def _fold(parts):
    return pl.pallas_call(
        _fold_body,
        out_shape=jax.ShapeDtypeStruct((NF, NB), jnp.float32),
    )(parts)


def kernel(x):
    mn2_sp, inv_sp = _minmax(x)
    parts = _sc_hist(x, mn2_sp, inv_sp)
    return _fold(parts.reshape(NW, NF, NBP))


# speedup vs baseline: 1.1860x; 1.1806x over previous
import functools

import jax
import jax.numpy as jnp
from jax import lax
from jax.experimental import pallas as pl
from jax.experimental.pallas import tpu as pltpu
from jax.experimental.pallas import tpu_sc as plsc

NF = 64
NB = 100
NBP = 112
BATCH = 2048
LANE = 256
NC, NS, L = 2, 16, 16
NW = NC * NS
BPW = BATCH // NW
CH = 2
NCHUNK = BPW // CH


_TC_CHUNK = 128


def _minmax_body(x_ref, mn2_ref, inv_ref, mn_ref, mx_ref):
    i = pl.program_id(0)
    blk = x_ref[...]
    bmn = jnp.min(blk, axis=(0, 2)).reshape(NF, 1)
    bmx = jnp.max(blk, axis=(0, 2)).reshape(NF, 1)

    @pl.when(i == 0)
    def _():
        mn_ref[...] = bmn
        mx_ref[...] = bmx

    @pl.when(i > 0)
    def _():
        mn_ref[...] = jnp.minimum(mn_ref[...], bmn)
        mx_ref[...] = jnp.maximum(mx_ref[...], bmx)

    @pl.when(i == pl.num_programs(0) - 1)
    def _():
        mn = mn_ref[...]
        w = (mx_ref[...] - mn) / NB
        frow = lax.broadcasted_iota(
            jnp.int32, (NF, 1), 0).astype(jnp.float32) * float(NBP)
        mn2 = mn - frow * w
        inv_ref[...] = jnp.broadcast_to(1.0 / w, (NF, L))
        mn2_ref[...] = jnp.broadcast_to(mn2, (NF, L))


def _minmax(x):
    return pl.pallas_call(
        _minmax_body,
        grid=(BATCH // _TC_CHUNK,),
        in_specs=[pl.BlockSpec((_TC_CHUNK, NF, LANE), lambda i: (i, 0, 0))],
        out_specs=[
            pl.BlockSpec((NF, L), lambda i: (0, 0)),
            pl.BlockSpec((NF, L), lambda i: (0, 0)),
        ],
        out_shape=[
            jax.ShapeDtypeStruct((NF, L), jnp.float32),
            jax.ShapeDtypeStruct((NF, L), jnp.float32),
        ],
        scratch_shapes=[
            pltpu.VMEM((NF, 1), jnp.float32),
            pltpu.VMEM((NF, 1), jnp.float32),
        ],
    )(x)


_sc_mesh = plsc.VectorSubcoreMesh(
    core_axis_name="c", subcore_axis_name="s", num_cores=NC, num_subcores=NS)


@functools.partial(
    pl.kernel,
    out_type=jax.ShapeDtypeStruct((NW, NF * NBP), jnp.int32),
    mesh=_sc_mesh,
    compiler_params=pltpu.CompilerParams(needs_layout_passes=False),
    scratch_types=[
        pltpu.VMEM((CH, NF, LANE), jnp.float32),
        pltpu.VMEM((CH, NF, LANE), jnp.float32),
        pltpu.VMEM((NF * NBP,), jnp.int32),
        pltpu.VMEM((NF, L), jnp.float32),
        pltpu.VMEM((NF, L), jnp.float32),
        pltpu.SemaphoreType.DMA,
        pltpu.SemaphoreType.DMA,
    ],
)
def _sc_hist(x_hbm, mn_hbm, inv_hbm, out_hbm,
             buf_a, buf_b, hist, mn_v, inv_v, sem_a, sem_b):
    wid = lax.axis_index("s") * NC + lax.axis_index("c")
    base = wid * BPW

    pltpu.sync_copy(mn_hbm, mn_v)
    pltpu.sync_copy(inv_hbm, inv_v)

    zeros16 = jnp.zeros((L,), jnp.int32)
    ones16 = jnp.ones((L,), jnp.int32)

    def _zero_row(r, _):
        hist[pl.ds(r * L, L)] = zeros16
        return 0

    lax.fori_loop(0, NF * NBP // L, _zero_row, 0)

    def _chunk_copy(ci, buf, sem):
        return pltpu.make_async_copy(
            x_hbm.at[pl.ds(base + ci * CH, CH)], buf, sem)

    def _process(buf):
        def _row(r, _):
            bb = r // (NF // 2)
            f0 = r - bb * (NF // 2)
            f1 = f0 + NF // 2
            mnv0 = mn_v[f0, :]
            invv0 = inv_v[f0, :]
            mnv1 = mn_v[f1, :]
            invv1 = inv_v[f1, :]

            @plsc.parallel_loop(0, LANE, step=L, unroll=LANE // L)
            def _k(off):
                v0 = buf[bb, f0, pl.ds(off, L)]
                b0 = ((v0 - mnv0) * invv0).astype(jnp.int32)
                plsc.addupdate_scatter(hist, [b0], ones16)
                v1 = buf[bb, f1, pl.ds(off, L)]
                b1 = ((v1 - mnv1) * invv1).astype(jnp.int32)
                plsc.addupdate_scatter(hist, [b1], ones16)

            return 0

        lax.fori_loop(0, CH * NF // 2, _row, 0)

    _chunk_copy(0, buf_a, sem_a).start()

    def _outer(g, _):
        c0 = 2 * g
        _chunk_copy(c0, buf_a, sem_a).wait()
        _chunk_copy(c0 + 1, buf_b, sem_b).start()
        _process(buf_a)
        _chunk_copy(c0 + 1, buf_b, sem_b).wait()

        @pl.when(c0 + 2 < NCHUNK)
        def _():
            _chunk_copy(c0 + 2, buf_a, sem_a).start()

        _process(buf_b)
        return 0

    lax.fori_loop(0, NCHUNK // 2, _outer, 0)

    pltpu.sync_copy(hist, out_hbm.at[wid])


def _fold_body(p_ref, o_ref):
    h = jnp.sum(p_ref[...], axis=0).astype(jnp.float32)
    kcol = lax.broadcasted_iota(jnp.int32, (NF, NB), 1)
    o_ref[...] = h[:, :NB] + jnp.where(kcol == NB - 1, h[:, NB:NB + 1], 0.0)


def _fold(parts):
    return pl.pallas_call(
        _fold_body,
        out_shape=jax.ShapeDtypeStruct((NF, NB), jnp.float32),
    )(parts)


def kernel(x):
    mn2_sp, inv_sp = _minmax(x)
    parts = _sc_hist(x, mn2_sp, inv_sp)
    return _fold(parts.reshape(NW, NF, NBP))


# speedup vs baseline: 1.3253x; 1.1175x over previous
import functools

import jax
import jax.numpy as jnp
from jax import lax
from jax.experimental import pallas as pl
from jax.experimental.pallas import tpu as pltpu
from jax.experimental.pallas import tpu_sc as plsc

NF = 64
NB = 100
NBP = 112
BATCH = 2048
LANE = 256
NC, NS, L = 2, 16, 16
NW = NC * NS
BPW = BATCH // NW
CH = 2
NCHUNK = BPW // CH


_TC_CHUNK = 128


def _minmax_body(x_ref, mn2_ref, inv_ref, mn_ref, mx_ref):
    i = pl.program_id(0)
    blk = x_ref[...]
    bmn = jnp.min(blk, axis=(0, 2)).reshape(NF, 1)
    bmx = jnp.max(blk, axis=(0, 2)).reshape(NF, 1)

    @pl.when(i == 0)
    def _():
        mn_ref[...] = bmn
        mx_ref[...] = bmx

    @pl.when(i > 0)
    def _():
        mn_ref[...] = jnp.minimum(mn_ref[...], bmn)
        mx_ref[...] = jnp.maximum(mx_ref[...], bmx)

    @pl.when(i == pl.num_programs(0) - 1)
    def _():
        mn = mn_ref[...]
        w = (mx_ref[...] - mn) / NB
        frow = lax.broadcasted_iota(
            jnp.int32, (NF, 1), 0).astype(jnp.float32) * float(NBP)
        mn2 = mn - frow * w
        inv_ref[...] = jnp.broadcast_to(1.0 / w, (NF, L))
        mn2_ref[...] = jnp.broadcast_to(mn2, (NF, L))


def _minmax(x):
    return pl.pallas_call(
        _minmax_body,
        grid=(BATCH // _TC_CHUNK,),
        in_specs=[pl.BlockSpec((_TC_CHUNK, NF, LANE), lambda i: (i, 0, 0))],
        out_specs=[
            pl.BlockSpec((NF, L), lambda i: (0, 0)),
            pl.BlockSpec((NF, L), lambda i: (0, 0)),
        ],
        out_shape=[
            jax.ShapeDtypeStruct((NF, L), jnp.float32),
            jax.ShapeDtypeStruct((NF, L), jnp.float32),
        ],
        scratch_shapes=[
            pltpu.VMEM((NF, 1), jnp.float32),
            pltpu.VMEM((NF, 1), jnp.float32),
        ],
    )(x)


_sc_mesh = plsc.VectorSubcoreMesh(
    core_axis_name="c", subcore_axis_name="s", num_cores=NC, num_subcores=NS)


@functools.partial(
    pl.kernel,
    out_type=jax.ShapeDtypeStruct((NW, NF * NBP), jnp.int32),
    mesh=_sc_mesh,
    compiler_params=pltpu.CompilerParams(needs_layout_passes=False),
    scratch_types=[
        pltpu.VMEM((CH, NF, LANE), jnp.float32),
        pltpu.VMEM((CH, NF, LANE), jnp.float32),
        pltpu.VMEM((NF * NBP,), jnp.int32),
        pltpu.VMEM((NF, L), jnp.float32),
        pltpu.VMEM((NF, L), jnp.float32),
        pltpu.SemaphoreType.DMA,
        pltpu.SemaphoreType.DMA,
    ],
)
def _sc_hist(x_hbm, mn_hbm, inv_hbm, out_hbm,
             buf_a, buf_b, hist, mn_v, inv_v, sem_a, sem_b):
    wid = lax.axis_index("s") * NC + lax.axis_index("c")
    base = wid * BPW

    pltpu.sync_copy(mn_hbm, mn_v)
    pltpu.sync_copy(inv_hbm, inv_v)

    zeros16 = jnp.zeros((L,), jnp.int32)
    ones16 = jnp.ones((L,), jnp.int32)

    def _zero_row(r, _):
        hist[pl.ds(r * L, L)] = zeros16
        return 0

    lax.fori_loop(0, NF * NBP // L, _zero_row, 0)

    def _chunk_copy(ci, buf, sem):
        return pltpu.make_async_copy(
            x_hbm.at[pl.ds(base + ci * CH, CH)], buf, sem)

    def _process(buf):
        @plsc.parallel_loop(0, CH * NF // 2, unroll=2)
        def _row(r):
            bb = r // (NF // 2)
            f0 = r - bb * (NF // 2)
            f1 = f0 + NF // 2
            mnv0 = mn_v[f0, :]
            invv0 = inv_v[f0, :]
            mnv1 = mn_v[f1, :]
            invv1 = inv_v[f1, :]

            @plsc.parallel_loop(0, LANE, step=L, unroll=LANE // L)
            def _k(off):
                v0 = buf[bb, f0, pl.ds(off, L)]
                b0 = ((v0 - mnv0) * invv0).astype(jnp.int32)
                plsc.addupdate_scatter(hist, [b0], ones16)
                v1 = buf[bb, f1, pl.ds(off, L)]
                b1 = ((v1 - mnv1) * invv1).astype(jnp.int32)
                plsc.addupdate_scatter(hist, [b1], ones16)

    _chunk_copy(0, buf_a, sem_a).start()

    def _outer(g, _):
        c0 = 2 * g
        _chunk_copy(c0, buf_a, sem_a).wait()
        _chunk_copy(c0 + 1, buf_b, sem_b).start()
        _process(buf_a)
        _chunk_copy(c0 + 1, buf_b, sem_b).wait()

        @pl.when(c0 + 2 < NCHUNK)
        def _():
            _chunk_copy(c0 + 2, buf_a, sem_a).start()

        _process(buf_b)
        return 0

    lax.fori_loop(0, NCHUNK // 2, _outer, 0)

    pltpu.sync_copy(hist, out_hbm.at[wid])


def _fold_body(p_ref, o_ref):
    h = jnp.sum(p_ref[...], axis=0).astype(jnp.float32)
    kcol = lax.broadcasted_iota(jnp.int32, (NF, NB), 1)
    o_ref[...] = h[:, :NB] + jnp.where(kcol == NB - 1, h[:, NB:NB + 1], 0.0)


def _fold(parts):
    return pl.pallas_call(
        _fold_body,
        out_shape=jax.ShapeDtypeStruct((NF, NB), jnp.float32),
    )(parts)


def kernel(x):
    mn2_sp, inv_sp = _minmax(x)
    parts = _sc_hist(x, mn2_sp, inv_sp)
    return _fold(parts.reshape(NW, NF, NBP))
